# Initial kernel scaffold; baseline (speedup 1.0000x reference)
import jax
import jax.numpy as jnp
from jax.experimental import pallas as pl


def kernel(pos, batch, params):
    raise NotImplementedError("write your pallas kernel here")


# speedup vs baseline: 1.2143x; 1.2143x over previous
import functools

import jax
import jax.numpy as jnp
from jax import lax
from jax.experimental import pallas as pl
from jax.experimental.pallas import tpu as pltpu

EPS = 1e-5
R = 0.2
K = 20

N = 10000
ROW_TILE = 1000


def _bn(z, L):
    return (z - L["rm"]) / jnp.sqrt(L["rv"] + EPS) * L["g"] + L["be"]


def _mlp_layer(x, L):
    return _bn(jax.nn.relu(x @ L["W"].T + L["b"]), L)


def _run_mlp(x, layers):
    for L in layers:
        x = _mlp_layer(x, L)
    return x


def _fps(x, n_samples):
    d0 = jnp.sum((x - x[0]) ** 2, axis=1)

    def step(dist, _):
        nxt = jnp.argmax(dist)
        nd = jnp.sum((x - x[nxt]) ** 2, axis=1)
        return jnp.minimum(dist, nd), nxt

    _, idxs = lax.scan(step, d0, None, length=n_samples - 1)
    return jnp.concatenate([jnp.zeros((1,), dtype=idxs.dtype), idxs])


def _dock_module(x, layers):
    n = x.shape[0]
    q = (n + 1) // 2
    xs = lax.stop_gradient(x)
    idx = _fps(xs, q)
    qpts = xs[idx]
    d2 = (jnp.sum(qpts * qpts, axis=1, keepdims=True) - 2.0 * (qpts @ xs.T)
          + jnp.sum(xs * xs, axis=1)[None, :])
    d2m = jnp.where(d2 <= R * R, d2, jnp.inf)
    vals, col = lax.top_k(-d2m, K)
    valid = jnp.isfinite(vals)
    row = jnp.broadcast_to(jnp.arange(q)[:, None], (q, K))
    col_f = col.reshape(-1)
    row_f = row.reshape(-1)
    valid_f = valid.reshape(-1)
    x_i = x[col_f]
    x_j = x[row_f]
    msg = _run_mlp(jnp.concatenate([x_i, x_j - x_i], axis=-1), layers)
    out = jnp.full((n, msg.shape[1]), -jnp.inf, dtype=msg.dtype)
    out = out.at[col_f].max(jnp.where(valid_f[:, None], msg, -jnp.inf))
    return jnp.where(jnp.isneginf(out), 0.0, out)


def _head_kernel(x_ref, w1_ref, p1_ref, w2_ref, p2_ref, w3_ref, p3_ref,
                 wf_ref, bf_ref, out_ref, gmax_ref):
    i = pl.program_id(0)
    nsteps = pl.num_programs(0)

    z = jnp.dot(x_ref[...], w1_ref[...], preferred_element_type=jnp.float32)
    b, g, be, rm, rv = (p1_ref[0:1, :], p1_ref[1:2, :], p1_ref[2:3, :],
                        p1_ref[3:4, :], p1_ref[4:5, :])
    h = (jax.nn.relu(z + b) - rm) / jnp.sqrt(rv + EPS) * g + be
    tile_max = jnp.max(h, axis=0, keepdims=True)

    @pl.when(i == 0)
    def _():
        gmax_ref[...] = tile_max

    @pl.when(i > 0)
    def _():
        gmax_ref[...] = jnp.maximum(gmax_ref[...], tile_max)

    @pl.when(i == nsteps - 1)
    def _():
        gv = gmax_ref[...]
        z2 = jnp.dot(gv, w2_ref[...], preferred_element_type=jnp.float32)
        b2, g2, be2, rm2, rv2 = (p2_ref[0:1, :], p2_ref[1:2, :], p2_ref[2:3, :],
                                 p2_ref[3:4, :], p2_ref[4:5, :])
        y = (jax.nn.relu(z2 + b2) - rm2) / jnp.sqrt(rv2 + EPS) * g2 + be2
        z3 = jnp.dot(y, w3_ref[...], preferred_element_type=jnp.float32)
        b3, g3, be3, rm3, rv3 = (p3_ref[0:1, :], p3_ref[1:2, :], p3_ref[2:3, :],
                                 p3_ref[3:4, :], p3_ref[4:5, :])
        y = (jax.nn.relu(z3 + b3) - rm3) / jnp.sqrt(rv3 + EPS) * g3 + be3
        logits = jnp.dot(y, wf_ref[...], preferred_element_type=jnp.float32)
        logits = logits + bf_ref[0:1, :]
        m = jnp.max(logits, axis=1, keepdims=True)
        lse = jnp.log(jnp.sum(jnp.exp(logits - m), axis=1, keepdims=True)) + m
        out_ref[...] = logits - lse


def _pack_bn(L):
    return jnp.stack([L["b"], L["g"], L["be"], L["rm"], L["rv"]], axis=0)


def _head(x, params):
    l1 = params["lin1"][0]
    m1, m2 = params["mlp"]
    n = x.shape[0]
    grid = n // ROW_TILE
    out = pl.pallas_call(
        _head_kernel,
        grid=(grid,),
        in_specs=[
            pl.BlockSpec((ROW_TILE, 192), lambda i: (i, 0)),
            pl.BlockSpec((192, 1024), lambda i: (0, 0)),
            pl.BlockSpec((5, 1024), lambda i: (0, 0)),
            pl.BlockSpec((1024, 512), lambda i: (0, 0)),
            pl.BlockSpec((5, 512), lambda i: (0, 0)),
            pl.BlockSpec((512, 256), lambda i: (0, 0)),
            pl.BlockSpec((5, 256), lambda i: (0, 0)),
            pl.BlockSpec((256, 40), lambda i: (0, 0)),
            pl.BlockSpec((1, 40), lambda i: (0, 0)),
        ],
        out_specs=pl.BlockSpec((1, 40), lambda i: (0, 0)),
        out_shape=jax.ShapeDtypeStruct((1, 40), jnp.float32),
        scratch_shapes=[pltpu.VMEM((1, 1024), jnp.float32)],
    )(x, l1["W"].T, _pack_bn(l1), m1["W"].T, _pack_bn(m1), m2["W"].T,
      _pack_bn(m2), params["final_W"].T, params["final_b"][None, :])
    return out


def kernel(pos, batch, params):
    x1 = _dock_module(pos, params["conv1"])
    x2 = _dock_module(x1, params["conv2"])
    x = jnp.concatenate([x1, x2], axis=1)
    return _head(x, params)


# speedup vs baseline: 5.0153x; 4.1302x over previous
import functools

import jax
import jax.numpy as jnp
from jax import lax
from jax.experimental import pallas as pl
from jax.experimental.pallas import tpu as pltpu

EPS = 1e-5
R = 0.2
K = 20

N = 10000
ROW_TILE = 1000


def _bn(z, L):
    return (z - L["rm"]) / jnp.sqrt(L["rv"] + EPS) * L["g"] + L["be"]


def _mlp_layer(x, L):
    return _bn(jax.nn.relu(x @ L["W"].T + L["b"]), L)


def _run_mlp(x, layers):
    for L in layers:
        x = _mlp_layer(x, L)
    return x


def _fps_kernel(xr_ref, x3_ref, q_ref, dist_ref, *, n, q_count, lanes):
    dim = xr_ref.shape[1]
    s_iota = lax.broadcasted_iota(jnp.int32, (8, lanes), 0)
    l_iota = lax.broadcasted_iota(jnp.int32, (8, lanes), 1)
    j_iota = s_iota * lanes + l_iota
    in_range = j_iota < n

    def new_dist(c_row):
        cb = c_row.reshape(dim, 1, 1)
        diff = x3_ref[...] - cb
        return jnp.sum(diff * diff, axis=0)

    c0 = xr_ref[0:1, :]
    q_ref[0:1, :] = c0
    dist_ref[...] = jnp.where(in_range, new_dist(c0), -jnp.inf)

    def body(t, _):
        dist = dist_ref[...]
        m = jnp.max(dist)
        nxt = jnp.min(jnp.where(dist == m, j_iota, jnp.int32(2**30)))
        c = xr_ref[pl.ds(nxt, 1), :]
        q_ref[pl.ds(t, 1), :] = c
        dist_ref[...] = jnp.minimum(dist, new_dist(c))
        return 0

    lax.fori_loop(1, q_count, body, 0)


def _fps_q(x, q_count):
    n, dim = x.shape
    lanes = ((n + 8 * 128 - 1) // (8 * 128)) * 128
    npad = 8 * lanes
    x_pad = jnp.zeros((npad, dim), x.dtype).at[:n].set(x)
    x3 = x_pad.T.reshape(dim, 8, lanes)
    return pl.pallas_call(
        functools.partial(_fps_kernel, n=n, q_count=q_count, lanes=lanes),
        in_specs=[
            pl.BlockSpec((npad, dim), lambda: (0, 0)),
            pl.BlockSpec((dim, 8, lanes), lambda: (0, 0, 0)),
        ],
        out_specs=pl.BlockSpec((q_count, dim), lambda: (0, 0)),
        out_shape=jax.ShapeDtypeStruct((q_count, dim), jnp.float32),
        scratch_shapes=[pltpu.VMEM((8, lanes), jnp.float32)],
    )(x_pad, x3)


def _dock_module(x, layers):
    n = x.shape[0]
    q = (n + 1) // 2
    xs = lax.stop_gradient(x)
    qpts = _fps_q(xs, q)
    d2 = (jnp.sum(qpts * qpts, axis=1, keepdims=True) - 2.0 * (qpts @ xs.T)
          + jnp.sum(xs * xs, axis=1)[None, :])
    d2m = jnp.where(d2 <= R * R, d2, jnp.inf)
    vals, col = lax.top_k(-d2m, K)
    valid = jnp.isfinite(vals)
    row = jnp.broadcast_to(jnp.arange(q)[:, None], (q, K))
    col_f = col.reshape(-1)
    row_f = row.reshape(-1)
    valid_f = valid.reshape(-1)
    x_i = x[col_f]
    x_j = x[row_f]
    msg = _run_mlp(jnp.concatenate([x_i, x_j - x_i], axis=-1), layers)
    out = jnp.full((n, msg.shape[1]), -jnp.inf, dtype=msg.dtype)
    out = out.at[col_f].max(jnp.where(valid_f[:, None], msg, -jnp.inf))
    return jnp.where(jnp.isneginf(out), 0.0, out)


def _head_kernel(x_ref, w1_ref, p1_ref, w2_ref, p2_ref, w3_ref, p3_ref,
                 wf_ref, bf_ref, out_ref, gmax_ref):
    i = pl.program_id(0)
    nsteps = pl.num_programs(0)

    z = jnp.dot(x_ref[...], w1_ref[...], preferred_element_type=jnp.float32)
    b, g, be, rm, rv = (p1_ref[0:1, :], p1_ref[1:2, :], p1_ref[2:3, :],
                        p1_ref[3:4, :], p1_ref[4:5, :])
    h = (jax.nn.relu(z + b) - rm) / jnp.sqrt(rv + EPS) * g + be
    tile_max = jnp.max(h, axis=0, keepdims=True)

    @pl.when(i == 0)
    def _():
        gmax_ref[...] = tile_max

    @pl.when(i > 0)
    def _():
        gmax_ref[...] = jnp.maximum(gmax_ref[...], tile_max)

    @pl.when(i == nsteps - 1)
    def _():
        gv = gmax_ref[...]
        z2 = jnp.dot(gv, w2_ref[...], preferred_element_type=jnp.float32)
        b2, g2, be2, rm2, rv2 = (p2_ref[0:1, :], p2_ref[1:2, :], p2_ref[2:3, :],
                                 p2_ref[3:4, :], p2_ref[4:5, :])
        y = (jax.nn.relu(z2 + b2) - rm2) / jnp.sqrt(rv2 + EPS) * g2 + be2
        z3 = jnp.dot(y, w3_ref[...], preferred_element_type=jnp.float32)
        b3, g3, be3, rm3, rv3 = (p3_ref[0:1, :], p3_ref[1:2, :], p3_ref[2:3, :],
                                 p3_ref[3:4, :], p3_ref[4:5, :])
        y = (jax.nn.relu(z3 + b3) - rm3) / jnp.sqrt(rv3 + EPS) * g3 + be3
        logits = jnp.dot(y, wf_ref[...], preferred_element_type=jnp.float32)
        logits = logits + bf_ref[0:1, :]
        m = jnp.max(logits, axis=1, keepdims=True)
        lse = jnp.log(jnp.sum(jnp.exp(logits - m), axis=1, keepdims=True)) + m
        out_ref[...] = logits - lse


def _pack_bn(L):
    return jnp.stack([L["b"], L["g"], L["be"], L["rm"], L["rv"]], axis=0)


def _head(x, params):
    l1 = params["lin1"][0]
    m1, m2 = params["mlp"]
    n = x.shape[0]
    grid = n // ROW_TILE
    out = pl.pallas_call(
        _head_kernel,
        grid=(grid,),
        in_specs=[
            pl.BlockSpec((ROW_TILE, 192), lambda i: (i, 0)),
            pl.BlockSpec((192, 1024), lambda i: (0, 0)),
            pl.BlockSpec((5, 1024), lambda i: (0, 0)),
            pl.BlockSpec((1024, 512), lambda i: (0, 0)),
            pl.BlockSpec((5, 512), lambda i: (0, 0)),
            pl.BlockSpec((512, 256), lambda i: (0, 0)),
            pl.BlockSpec((5, 256), lambda i: (0, 0)),
            pl.BlockSpec((256, 40), lambda i: (0, 0)),
            pl.BlockSpec((1, 40), lambda i: (0, 0)),
        ],
        out_specs=pl.BlockSpec((1, 40), lambda i: (0, 0)),
        out_shape=jax.ShapeDtypeStruct((1, 40), jnp.float32),
        scratch_shapes=[pltpu.VMEM((1, 1024), jnp.float32)],
    )(x, l1["W"].T, _pack_bn(l1), m1["W"].T, _pack_bn(m1), m2["W"].T,
      _pack_bn(m2), params["final_W"].T, params["final_b"][None, :])
    return out


def kernel(pos, batch, params):
    x1 = _dock_module(pos, params["conv1"])
    x2 = _dock_module(x1, params["conv2"])
    x = jnp.concatenate([x1, x2], axis=1)
    return _head(x, params)


# speedup vs baseline: 9.8900x; 1.9720x over previous
import functools

import jax
import jax.numpy as jnp
from jax import lax
from jax.experimental import pallas as pl
from jax.experimental.pallas import tpu as pltpu

EPS = 1e-5
R = 0.2
K = 20

N = 10000
ROW_TILE = 1000


def _bn(z, L):
    return (z - L["rm"]) / jnp.sqrt(L["rv"] + EPS) * L["g"] + L["be"]


def _mlp_layer(x, L):
    return _bn(jax.nn.relu(x @ L["W"].T + L["b"]), L)


def _run_mlp(x, layers):
    for L in layers:
        x = _mlp_layer(x, L)
    return x


def _fps_kernel(xr_ref, x3_ref, q_ref, dist_ref, *, n, q_count, lanes):
    dim = xr_ref.shape[1]
    s_iota = lax.broadcasted_iota(jnp.int32, (8, lanes), 0)
    l_iota = lax.broadcasted_iota(jnp.int32, (8, lanes), 1)
    j_iota = s_iota * lanes + l_iota
    in_range = j_iota < n

    def new_dist(c_row):
        cb = c_row.reshape(dim, 1, 1)
        diff = x3_ref[...] - cb
        return jnp.sum(diff * diff, axis=0)

    c0 = xr_ref[0:1, :]
    q_ref[0:1, :] = c0
    dist_ref[...] = jnp.where(in_range, new_dist(c0), -jnp.inf)

    def body(t, _):
        dist = dist_ref[...]
        m = jnp.max(dist)
        nxt = jnp.min(jnp.where(dist == m, j_iota, jnp.int32(2**30)))
        c = xr_ref[pl.ds(nxt, 1), :]
        q_ref[pl.ds(t, 1), :] = c
        dist_ref[...] = jnp.minimum(dist, new_dist(c))
        return 0

    lax.fori_loop(1, q_count, body, 0)


def _fps_q(x, q_count):
    n, dim = x.shape
    lanes = ((n + 8 * 128 - 1) // (8 * 128)) * 128
    npad = 8 * lanes
    x_pad = jnp.zeros((npad, dim), x.dtype).at[:n].set(x)
    x3 = x_pad.T.reshape(dim, 8, lanes)
    return pl.pallas_call(
        functools.partial(_fps_kernel, n=n, q_count=q_count, lanes=lanes),
        in_specs=[
            pl.BlockSpec((npad, dim), lambda: (0, 0)),
            pl.BlockSpec((dim, 8, lanes), lambda: (0, 0, 0)),
        ],
        out_specs=pl.BlockSpec((q_count, dim), lambda: (0, 0)),
        out_shape=jax.ShapeDtypeStruct((q_count, dim), jnp.float32),
        scratch_shapes=[pltpu.VMEM((8, lanes), jnp.float32)],
    )(x_pad, x3)


def _topk_kernel(q_ref, xt_ref, col_ref, val_ref, *, n):
    dim, npad = xt_ref.shape
    xt = xt_ref[...]
    sq = jnp.sum(xt * xt, axis=0, keepdims=True)
    qv = q_ref[...]
    qq = jnp.sum(qv * qv, axis=1, keepdims=True)
    dot = jnp.dot(qv, xt, preferred_element_type=jnp.float32,
                  precision=lax.Precision.DEFAULT)
    d2 = qq - 2.0 * dot + sq
    col_iota = lax.broadcasted_iota(jnp.int32, d2.shape, 1)
    d2m = jnp.where((d2 <= R * R) & (col_iota < n), d2, jnp.inf)
    for r in range(K):
        m = jnp.min(d2m, axis=1, keepdims=True)
        sel = jnp.min(jnp.where(d2m == m, col_iota, jnp.int32(2**30)),
                      axis=1, keepdims=True)
        col_ref[:, r:r + 1] = sel
        val_ref[:, r:r + 1] = jnp.where(jnp.isfinite(m), 1.0, 0.0)
        if r < K - 1:
            d2m = jnp.where(col_iota == sel, jnp.inf, d2m)


def _radius_topk(qpts, x):
    qn, dim = qpts.shape
    n = x.shape[0]
    tq = 128
    qpad = ((qn + tq - 1) // tq) * tq
    npad = ((n + 127) // 128) * 128
    q_p = jnp.zeros((qpad, dim), qpts.dtype).at[:qn].set(qpts)
    xt = jnp.zeros((dim, npad), x.dtype).at[:, :n].set(x.T)
    col, val = pl.pallas_call(
        functools.partial(_topk_kernel, n=n),
        grid=(qpad // tq,),
        in_specs=[
            pl.BlockSpec((tq, dim), lambda i: (i, 0)),
            pl.BlockSpec((dim, npad), lambda i: (0, 0)),
        ],
        out_specs=[
            pl.BlockSpec((tq, K), lambda i: (i, 0)),
            pl.BlockSpec((tq, K), lambda i: (i, 0)),
        ],
        out_shape=[
            jax.ShapeDtypeStruct((qpad, K), jnp.int32),
            jax.ShapeDtypeStruct((qpad, K), jnp.float32),
        ],
    )(q_p, xt)
    return col[:qn], val[:qn]


def _dock_module(x, layers):
    n = x.shape[0]
    q = (n + 1) // 2
    xs = lax.stop_gradient(x)
    qpts = _fps_q(xs, q)
    col, valid = _radius_topk(qpts, xs)
    row = jnp.broadcast_to(jnp.arange(q)[:, None], (q, K))
    col_f = jnp.minimum(col.reshape(-1), n - 1)
    row_f = row.reshape(-1)
    valid_f = valid.reshape(-1) > 0.5
    x_i = x[col_f]
    x_j = x[row_f]
    msg = _run_mlp(jnp.concatenate([x_i, x_j - x_i], axis=-1), layers)
    out = jnp.full((n, msg.shape[1]), -jnp.inf, dtype=msg.dtype)
    out = out.at[col_f].max(jnp.where(valid_f[:, None], msg, -jnp.inf))
    return jnp.where(jnp.isneginf(out), 0.0, out)


def _head_kernel(x_ref, w1_ref, p1_ref, w2_ref, p2_ref, w3_ref, p3_ref,
                 wf_ref, bf_ref, out_ref, gmax_ref):
    i = pl.program_id(0)
    nsteps = pl.num_programs(0)

    z = jnp.dot(x_ref[...], w1_ref[...], preferred_element_type=jnp.float32)
    b, g, be, rm, rv = (p1_ref[0:1, :], p1_ref[1:2, :], p1_ref[2:3, :],
                        p1_ref[3:4, :], p1_ref[4:5, :])
    h = (jax.nn.relu(z + b) - rm) / jnp.sqrt(rv + EPS) * g + be
    tile_max = jnp.max(h, axis=0, keepdims=True)

    @pl.when(i == 0)
    def _():
        gmax_ref[...] = tile_max

    @pl.when(i > 0)
    def _():
        gmax_ref[...] = jnp.maximum(gmax_ref[...], tile_max)

    @pl.when(i == nsteps - 1)
    def _():
        gv = gmax_ref[...]
        z2 = jnp.dot(gv, w2_ref[...], preferred_element_type=jnp.float32)
        b2, g2, be2, rm2, rv2 = (p2_ref[0:1, :], p2_ref[1:2, :], p2_ref[2:3, :],
                                 p2_ref[3:4, :], p2_ref[4:5, :])
        y = (jax.nn.relu(z2 + b2) - rm2) / jnp.sqrt(rv2 + EPS) * g2 + be2
        z3 = jnp.dot(y, w3_ref[...], preferred_element_type=jnp.float32)
        b3, g3, be3, rm3, rv3 = (p3_ref[0:1, :], p3_ref[1:2, :], p3_ref[2:3, :],
                                 p3_ref[3:4, :], p3_ref[4:5, :])
        y = (jax.nn.relu(z3 + b3) - rm3) / jnp.sqrt(rv3 + EPS) * g3 + be3
        logits = jnp.dot(y, wf_ref[...], preferred_element_type=jnp.float32)
        logits = logits + bf_ref[0:1, :]
        m = jnp.max(logits, axis=1, keepdims=True)
        lse = jnp.log(jnp.sum(jnp.exp(logits - m), axis=1, keepdims=True)) + m
        out_ref[...] = logits - lse


def _pack_bn(L):
    return jnp.stack([L["b"], L["g"], L["be"], L["rm"], L["rv"]], axis=0)


def _head(x, params):
    l1 = params["lin1"][0]
    m1, m2 = params["mlp"]
    n = x.shape[0]
    grid = n // ROW_TILE
    out = pl.pallas_call(
        _head_kernel,
        grid=(grid,),
        in_specs=[
            pl.BlockSpec((ROW_TILE, 192), lambda i: (i, 0)),
            pl.BlockSpec((192, 1024), lambda i: (0, 0)),
            pl.BlockSpec((5, 1024), lambda i: (0, 0)),
            pl.BlockSpec((1024, 512), lambda i: (0, 0)),
            pl.BlockSpec((5, 512), lambda i: (0, 0)),
            pl.BlockSpec((512, 256), lambda i: (0, 0)),
            pl.BlockSpec((5, 256), lambda i: (0, 0)),
            pl.BlockSpec((256, 40), lambda i: (0, 0)),
            pl.BlockSpec((1, 40), lambda i: (0, 0)),
        ],
        out_specs=pl.BlockSpec((1, 40), lambda i: (0, 0)),
        out_shape=jax.ShapeDtypeStruct((1, 40), jnp.float32),
        scratch_shapes=[pltpu.VMEM((1, 1024), jnp.float32)],
    )(x, l1["W"].T, _pack_bn(l1), m1["W"].T, _pack_bn(m1), m2["W"].T,
      _pack_bn(m2), params["final_W"].T, params["final_b"][None, :])
    return out


def kernel(pos, batch, params):
    x1 = _dock_module(pos, params["conv1"])
    x2 = _dock_module(x1, params["conv2"])
    x = jnp.concatenate([x1, x2], axis=1)
    return _head(x, params)
